# Initial kernel scaffold; baseline (speedup 1.0000x reference)
import jax
import jax.numpy as jnp
from jax.experimental import pallas as pl


def kernel(x, params):
    raise NotImplementedError("write your pallas kernel here")


# speedup vs baseline: 3.8830x; 3.8830x over previous
import jax
import jax.numpy as jnp
from jax.experimental import pallas as pl
from jax.experimental.pallas import tpu as pltpu

H = 256
HD = 128
NM = 3
NL = 3
B = 32
L = 50
N = 64

ROWS = B * N
TILE = 256
NTILES = ROWS // TILE


def _gates(g):
    i = jax.nn.sigmoid(g[:, :HD])
    f = jax.nn.sigmoid(g[:, HD:2 * HD])
    gg = jnp.tanh(g[:, 2 * HD:3 * HD])
    o = jax.nn.sigmoid(g[:, 3 * HD:])
    return i, f, gg, o


def _lstm_kernel(x_ref,
                 wih0f_ref, whh0f_ref, b0f_ref,
                 wih0r_ref, whh0r_ref, b0r_ref,
                 wih1f_ref, whh1f_ref, b1f_ref,
                 wih1r_ref, whh1r_ref, b1r_ref,
                 out_ref, h0r_store, xs):
    R = x_ref.shape[0]
    zero = jnp.zeros((R, HD), dtype=jnp.float32)

    for t in range(L):
        xs[t] = x_ref[:, 2 * t:2 * t + 2]

    w0f0 = wih0f_ref[0:1, :]
    w0f1 = wih0f_ref[1:2, :]
    w0r0 = wih0r_ref[0:1, :]
    w0r1 = wih0r_ref[1:2, :]
    b0f = b0f_ref[0:1, :]
    b0r = b0r_ref[0:1, :]
    b1f = b1f_ref[0:1, :]
    b1r = b1r_ref[0:1, :]

    def xt(t):
        pair = xs[t]
        return pair[:, 0:1], pair[:, 1:2]

    def rev_body(k, carry):
        h, c = carry
        t = L - 1 - k
        x0, x1 = xt(t)
        g = (x0 * w0r0 + x1 * w0r1 + b0r
             + jnp.dot(h, whh0r_ref[...], preferred_element_type=jnp.float32))
        i, f, gg, o = _gates(g)
        c = f * c + i * gg
        h = o * jnp.tanh(c)
        h0r_store[t] = h
        return h, c

    jax.lax.fori_loop(0, L, rev_body, (zero, zero))

    def fwd_body(t, carry):
        h0, c0, h1, c1 = carry
        x0, x1 = xt(t)
        g0 = (x0 * w0f0 + x1 * w0f1 + b0f
              + jnp.dot(h0, whh0f_ref[...], preferred_element_type=jnp.float32))
        i, f, gg, o = _gates(g0)
        c0 = f * c0 + i * gg
        h0 = o * jnp.tanh(c0)
        h0cat = jnp.concatenate([h0, h0r_store[t]], axis=1)
        g1 = (jnp.dot(h0cat, wih1f_ref[...], preferred_element_type=jnp.float32)
              + b1f
              + jnp.dot(h1, whh1f_ref[...], preferred_element_type=jnp.float32))
        i, f, gg, o = _gates(g1)
        c1 = f * c1 + i * gg
        h1 = o * jnp.tanh(c1)
        return h0, c0, h1, c1

    h0, _, h1f, _ = jax.lax.fori_loop(0, L, fwd_body, (zero, zero, zero, zero))

    h0last = jnp.concatenate([h0, h0r_store[L - 1]], axis=1)
    g = (jnp.dot(h0last, wih1r_ref[...], preferred_element_type=jnp.float32)
         + b1r)
    i, f, gg, o = _gates(g)
    c = i * gg
    h1r = o * jnp.tanh(c)

    out_ref[:, :HD] = h1f
    out_ref[:, HD:] = h1r


def _ln_global(v, w, b, eps=1e-5):
    m = jnp.mean(v)
    var = jnp.mean((v - m) ** 2)
    return (v - m) * jax.lax.rsqrt(var + eps) * w + b


def _gnn_kernel(last_ref,
                t0_ref, w10_ref, b10_ref, lnw0_ref, lnb0_ref,
                w20_ref, b20_ref, nw0_ref, nb0_ref,
                t1_ref, w11_ref, b11_ref, lnw1_ref, lnb1_ref,
                w21_ref, b21_ref, nw1_ref, nb1_ref,
                t2_ref, w12_ref, b12_ref, lnw2_ref, lnb2_ref,
                w22_ref, b22_ref, nw2_ref, nb2_ref,
                fc1_ref, fc1b_ref, fc_ref, fcb_ref,
                out_ref):
    h = last_ref[...]
    rows = jax.lax.broadcasted_iota(jnp.int32, (ROWS, 1), 0)
    in_graph = rows < N

    layers = [
        (t0_ref, w10_ref, b10_ref, lnw0_ref, lnb0_ref, w20_ref, b20_ref,
         nw0_ref, nb0_ref),
        (t1_ref, w11_ref, b11_ref, lnw1_ref, lnb1_ref, w21_ref, b21_ref,
         nw1_ref, nb1_ref),
        (t2_ref, w12_ref, b12_ref, lnw2_ref, lnb2_ref, w22_ref, b22_ref,
         nw2_ref, nb2_ref),
    ]
    for (t_ref, w1_ref, b1_ref, lnw_ref, lnb_ref, w2_ref, b2_ref,
         nw_ref, nb_ref) in layers:
        msg = jax.nn.relu(h[:N, :]) + 1e-7
        logits = msg * t_ref[0, 0]
        smax = jnp.max(logits, axis=0, keepdims=True)
        ex = jnp.exp(logits - smax)
        den = jnp.sum(ex, axis=0, keepdims=True)
        agg = jnp.sum(ex * msg, axis=0, keepdims=True) / (den + 1e-16)
        out = h + jnp.where(in_graph, agg, 0.0)
        v = jnp.dot(out, w1_ref[...], preferred_element_type=jnp.float32)
        v = v + b1_ref[0:1, :]
        v = _ln_global(v, lnw_ref[0:1, :], lnb_ref[0:1, :])
        v = jax.nn.relu(v)
        v = jnp.dot(v, w2_ref[...], preferred_element_type=jnp.float32)
        v = v + b2_ref[0:1, :]
        v = _ln_global(v, nw_ref[0:1, :], nb_ref[0:1, :])
        h = jax.nn.relu(v)

    v = jnp.dot(h, fc1_ref[...], preferred_element_type=jnp.float32)
    v = jax.nn.relu(v + fc1b_ref[0:1, :])
    gp = jnp.dot(v, fc_ref[...], preferred_element_type=jnp.float32)
    gp = gp + fcb_ref[0:1, :]

    a = gp[:, 0:1]
    b = gp[:, 1:2]
    c = gp[:, 2:3]
    m = jnp.maximum(jnp.maximum(a, b), c)
    e = jnp.exp(gp - m)
    s = e[:, 0:1] + e[:, 1:2] + e[:, 2:3]
    col = jax.lax.broadcasted_iota(jnp.int32, gp.shape, 1)
    out = jnp.where(col < NM, e / s,
                    jnp.where(col < 3 * NM, gp,
                              jnp.where(col < 5 * NM, jnp.exp(gp),
                                        jnp.where(col < 6 * NM,
                                                  jnp.tanh(gp), 0.0))))
    out_ref[...] = out


def _rep(shape):
    nd = len(shape)
    return pl.BlockSpec(shape, lambda i: (0,) * nd)


def kernel(x, params):
    p = params
    f32 = jnp.float32

    xi = jnp.transpose(x, (0, 2, 1, 3)).reshape(ROWS, L * 2)

    def bias(pref):
        return (p[pref + 'bih_f'] + p[pref + 'bhh_f']).reshape(1, 4 * HD), \
               (p[pref + 'bih_r'] + p[pref + 'bhh_r']).reshape(1, 4 * HD)

    b0f, b0r = bias('l0_')
    b1f, b1r = bias('l1_')
    lstm_args = (
        xi,
        p['l0_wih_f'].T, p['l0_whh_f'].T, b0f,
        p['l0_wih_r'].T, p['l0_whh_r'].T, b0r,
        p['l1_wih_f'].T, p['l1_whh_f'].T, b1f,
        p['l1_wih_r'].T, p['l1_whh_r'].T, b1r,
    )
    lstm_specs = [pl.BlockSpec((TILE, L * 2), lambda i: (i, 0))] + [
        _rep(a.shape) for a in lstm_args[1:]
    ]
    last = pl.pallas_call(
        _lstm_kernel,
        grid=(NTILES,),
        in_specs=lstm_specs,
        out_specs=pl.BlockSpec((TILE, 2 * HD), lambda i: (i, 0)),
        out_shape=jax.ShapeDtypeStruct((ROWS, 2 * HD), f32),
        scratch_shapes=[pltpu.VMEM((L, TILE, HD), f32),
                        pltpu.VMEM((L, TILE, 2), f32)],
    )(*lstm_args)

    PAD = 128

    def row(v):
        return v.reshape(1, -1)

    gnn_args = [last]
    for i in range(NL):
        gnn_args += [
            p['gcn%d_t' % i].reshape(1, 1),
            p['gcn%d_mlp_w1' % i].T, row(p['gcn%d_mlp_b1' % i]),
            row(p['gcn%d_mlp_ln_w' % i]), row(p['gcn%d_mlp_ln_b' % i]),
            p['gcn%d_mlp_w2' % i].T, row(p['gcn%d_mlp_b2' % i]),
            row(p['gcn%d_norm_w' % i]), row(p['gcn%d_norm_b' % i]),
        ]
    fcT = jnp.zeros((H, PAD), f32).at[:, :6 * NM].set(p['fc_w'].T)
    fcb = jnp.zeros((1, PAD), f32).at[:, :6 * NM].set(p['fc_b'])
    gnn_args += [p['fc1_w'].T, row(p['fc1_b']), fcT, fcb]

    gp = pl.pallas_call(
        _gnn_kernel,
        grid=(1,),
        in_specs=[_rep(a.shape) for a in gnn_args],
        out_specs=_rep((ROWS, PAD)),
        out_shape=jax.ShapeDtypeStruct((ROWS, PAD), f32),
    )(*gnn_args)

    gp = gp.reshape(B, N, PAD)
    pi = gp[..., :NM]
    mu = gp[..., NM:3 * NM]
    sigma = gp[..., 3 * NM:5 * NM]
    corr = gp[..., 5 * NM:6 * NM]
    return pi, mu, sigma, corr
